# baseline (speedup 1.0000x reference)
import functools

import jax
import jax.numpy as jnp
from jax import lax
from jax.experimental import pallas as pl
from jax.experimental.pallas import tpu as pltpu
from jax.experimental.pallas import tpu_sc as plsc

D_MODEL = 768
SEQ_LEN = 8192
NUM_CORES = 2
NUM_SUBCORES = 16
NUM_WORKERS = NUM_CORES * NUM_SUBCORES
ROWS_PER_WORKER = SEQ_LEN // NUM_WORKERS
CHUNK_SIZES = (16, 16, 32, 32, 32, 32, 32, 32, 16, 16)
CHUNK_OFFS = tuple(sum(CHUNK_SIZES[:i]) for i in range(len(CHUNK_SIZES)))
NUM_CHUNKS = len(CHUNK_SIZES)
BUF_ROWS = max(CHUNK_SIZES)
NBUF = 5

_mesh = plsc.VectorSubcoreMesh(core_axis_name="c", subcore_axis_name="s")


@functools.partial(
    pl.kernel,
    mesh=_mesh,
    out_type=jax.ShapeDtypeStruct((SEQ_LEN, D_MODEL), jnp.float32),
    scratch_types=(
        [pltpu.VMEM((ROWS_PER_WORKER,), jnp.int32)]
        + [pltpu.VMEM((BUF_ROWS, D_MODEL), jnp.float32)] * NBUF
        + [pltpu.SemaphoreType.DMA] * (2 * NBUF)
    ),
)
def _gather_kernel(table_hbm, idx_hbm, out_hbm, idx_v, *bufs_and_sems):
    bufs = bufs_and_sems[:NBUF]
    gsems = bufs_and_sems[NBUF:2 * NBUF]
    osems = bufs_and_sems[2 * NBUF:]

    wid = lax.axis_index("s") * NUM_CORES + lax.axis_index("c")
    base = wid * ROWS_PER_WORKER

    pltpu.sync_copy(idx_hbm.at[pl.ds(base, ROWS_PER_WORKER)], idx_v)

    gather = [None] * NBUF
    writeback = [None] * NBUF

    for c in range(NBUF):
        gather[c] = pltpu.async_copy(
            table_hbm.at[idx_v.at[pl.ds(CHUNK_OFFS[c], CHUNK_SIZES[c])]],
            bufs[c].at[pl.ds(0, CHUNK_SIZES[c])], gsems[c])

    for c in range(NUM_CHUNKS):
        i = c % NBUF
        gather[i].wait()
        writeback[i] = pltpu.async_copy(
            bufs[i].at[pl.ds(0, CHUNK_SIZES[c])],
            out_hbm.at[pl.ds(base + CHUNK_OFFS[c], CHUNK_SIZES[c])], osems[i])
        nxt = c + NBUF
        if nxt < NUM_CHUNKS:
            writeback[i].wait()
            gather[i] = pltpu.async_copy(
                table_hbm.at[idx_v.at[pl.ds(CHUNK_OFFS[nxt], CHUNK_SIZES[nxt])]],
                bufs[i].at[pl.ds(0, CHUNK_SIZES[nxt])], gsems[i])

    for i in range(NBUF):
        writeback[i].wait()


def kernel(positions, embedding_table):
    idx = jnp.asarray(positions, jnp.int32)
    return _gather_kernel(embedding_table, idx)


# speedup vs baseline: 1.0145x; 1.0145x over previous
import functools

import jax
import jax.numpy as jnp
from jax import lax
from jax.experimental import pallas as pl
from jax.experimental.pallas import tpu as pltpu
from jax.experimental.pallas import tpu_sc as plsc

D_MODEL = 768
SEQ_LEN = 8192
NUM_CORES = 2
NUM_SUBCORES = 16
NUM_WORKERS = NUM_CORES * NUM_SUBCORES
ROWS_PER_WORKER = SEQ_LEN // NUM_WORKERS
CHUNK = 32
NUM_CHUNKS = ROWS_PER_WORKER // CHUNK
NBUF = 5

_mesh = plsc.VectorSubcoreMesh(core_axis_name="c", subcore_axis_name="s")


@functools.partial(
    pl.kernel,
    mesh=_mesh,
    out_type=jax.ShapeDtypeStruct((SEQ_LEN, D_MODEL), jnp.float32),
    scratch_types=(
        [pltpu.VMEM((ROWS_PER_WORKER,), jnp.int32)]
        + [pltpu.VMEM((CHUNK, D_MODEL), jnp.float32)] * NBUF
        + [pltpu.SemaphoreType.DMA] * (2 * NBUF)
    ),
)
def _gather_kernel(table_hbm, idx_hbm, out_hbm, idx_v, *bufs_and_sems):
    bufs = bufs_and_sems[:NBUF]
    gsems = bufs_and_sems[NBUF:2 * NBUF]
    osems = bufs_and_sems[2 * NBUF:]

    wid = lax.axis_index("s") * NUM_CORES + lax.axis_index("c")
    base = wid * ROWS_PER_WORKER

    pltpu.sync_copy(idx_hbm.at[pl.ds(base, ROWS_PER_WORKER)], idx_v)

    gather = [None] * NBUF
    writeback = [None] * NBUF

    for c in range(NBUF):
        gather[c] = pltpu.async_copy(
            table_hbm.at[idx_v.at[pl.ds(c * CHUNK, CHUNK)]], bufs[c], gsems[c])

    for c in range(NUM_CHUNKS):
        i = c % NBUF
        gather[i].wait()
        writeback[i] = pltpu.async_copy(
            bufs[i], out_hbm.at[pl.ds(base + c * CHUNK, CHUNK)], osems[i])
        nxt = c + NBUF
        if nxt < NUM_CHUNKS:
            writeback[i].wait()
            gather[i] = pltpu.async_copy(
                table_hbm.at[idx_v.at[pl.ds(nxt * CHUNK, CHUNK)]],
                bufs[i], gsems[i])

    for i in range(NBUF):
        writeback[i].wait()


def kernel(positions, embedding_table):
    idx = jnp.asarray(positions, jnp.int32)
    return _gather_kernel(embedding_table, idx)
